# Initial kernel scaffold; baseline (speedup 1.0000x reference)
import jax
import jax.numpy as jnp
from jax.experimental import pallas as pl


def kernel(pillar_features, voxel_coords):
    raise NotImplementedError("write your pallas kernel here")


# speedup vs baseline: 3.8263x; 3.8263x over previous
import functools

import jax
import jax.numpy as jnp
from jax import lax
from jax.experimental import pallas as pl
from jax.experimental.pallas import tpu as pltpu
from jax.experimental.pallas import tpu_sc as plsc

B = 4
C = 64
NX = 432
NY = 496
M = 80000
NSLOT = B * NY * NX
NC = 2
NS = 16
NW = NC * NS
MP = M // NW
MPAD = 2512
NVEC = MPAD // 16
K = NSLOT // NW
TRASH = NSLOT
CANVAS_ROWS = NSLOT + 864
NCH = 20
CHW = 128
MROW = NCH * CHW

_mesh = plsc.VectorSubcoreMesh(
    core_axis_name="c", subcore_axis_name="s", num_cores=NC, num_subcores=NS)


def _wid():
    return lax.axis_index("s") * NC + lax.axis_index("c")


def _iota():
    return lax.iota(jnp.int32, 16)


@functools.partial(
    pl.kernel,
    out_type=jax.ShapeDtypeStruct((NW * MPAD,), jnp.int32),
    mesh=_mesh,
    compiler_params=pltpu.CompilerParams(
        use_tc_tiling_on_sc=False, needs_layout_passes=False),
    scratch_types=[
        pltpu.VMEM((MP, 4), jnp.int32),
        pltpu.VMEM((MPAD,), jnp.int32),
    ],
)
def _sc_flatten(coords_hbm, flat_hbm, coords_v, flat_v):
    w = _wid()
    pltpu.sync_copy(coords_hbm.at[w], coords_v)
    zeros = jnp.zeros((16,), jnp.int32)
    col_y = jnp.full((16,), 2, jnp.int32)
    col_x = jnp.full((16,), 3, jnp.int32)

    def body(i, _):
        p = i * 16 + _iota()
        valid = p < MP
        pc = jnp.minimum(p, MP - 1)
        b = plsc.load_gather(coords_v, [pc, zeros])
        y = plsc.load_gather(coords_v, [pc, col_y])
        x = plsc.load_gather(coords_v, [pc, col_x])
        f = b * (NY * NX) + y * NX + x
        flat_v[pl.ds(i * 16, 16)] = jnp.where(valid, f, -1)
        return 0

    lax.fori_loop(0, NVEC, body, 0)
    pltpu.sync_copy(flat_v, flat_hbm.at[pl.ds(w * MPAD, MPAD)])


@functools.partial(
    pl.kernel,
    out_type=jax.ShapeDtypeStruct((NSLOT,), jnp.int32),
    mesh=_mesh,
    compiler_params=pltpu.CompilerParams(
        use_tc_tiling_on_sc=False, needs_layout_passes=False),
    scratch_types=[
        pltpu.VMEM((NW * MPAD,), jnp.int32),
        pltpu.VMEM((K,), jnp.int32),
    ],
)
def _sc_winner(flat_hbm, winner_hbm, flat_v, seg_v):
    w = _wid()
    base = w * K
    pltpu.sync_copy(flat_hbm, flat_v)
    neg1 = jnp.full((16,), -1, jnp.int32)

    def init(i, _):
        seg_v[pl.ds(i * 16, 16)] = neg1
        return 0

    lax.fori_loop(0, K // 16, init, 0)

    def scan(j, _):
        f = flat_v[pl.ds(j * 16, 16)]
        local = f - base
        inr = local.astype(jnp.uint32) < jnp.uint32(K)
        cl = jnp.where(inr, local, 0)
        m = j * 16 + _iota()
        cur = plsc.load_gather(seg_v, [cl], mask=inr)
        new = jnp.maximum(cur, m)
        plsc.store_scatter(seg_v, [cl], new, mask=inr)
        return 0

    lax.fori_loop(0, (NW * MPAD) // 16, scan, 0)
    pltpu.sync_copy(seg_v, winner_hbm.at[pl.ds(base, K)])


@functools.partial(
    pl.kernel,
    out_type=jax.ShapeDtypeStruct((CANVAS_ROWS, C), jnp.float32),
    mesh=_mesh,
    compiler_params=pltpu.CompilerParams(
        use_tc_tiling_on_sc=False, needs_layout_passes=False),
    scratch_types=[
        pltpu.VMEM((MROW,), jnp.int32),
        pltpu.VMEM((NCH, CHW), jnp.int32),
        pltpu.VMEM((NCH, CHW), jnp.int32),
        pltpu.VMEM((NCH, CHW), jnp.int32),
        pltpu.VMEM((CHW, C), jnp.float32),
        pltpu.SemaphoreType.DMA,
    ],
)
def _sc_scatter(pf_hbm, flat_hbm, winner_hbm, canvas_hbm,
                fidx_v, gidx_v, wv_v, tgt_v, rows_v, sem):
    w = _wid()
    fbase = w * MPAD
    pltpu.sync_copy(flat_hbm.at[pl.ds(fbase, MPAD)], fidx_v.at[pl.ds(0, MPAD)])

    def sanitize(i, _):
        p = i * 16 + _iota()
        valid = p < MP
        f = fidx_v[pl.ds(i * 16, 16)]
        fs = jnp.where(valid, f, 0)
        plsc.store_scatter(gidx_v, [p // CHW, p % CHW], fs)
        return 0

    lax.fori_loop(0, MROW // 16, sanitize, 0)

    def gather(c, _):
        pltpu.async_copy(winner_hbm.at[gidx_v.at[c]], wv_v.at[c], sem).wait()
        return 0

    lax.fori_loop(0, NCH, gather, 0)

    def compare(i, _):
        p = i * 16 + _iota()
        valid = p < MP
        f = fidx_v[pl.ds(i * 16, 16)]
        wv = plsc.load_gather(wv_v, [p // CHW, p % CHW])
        win = (wv == (fbase + p)) & valid
        tgt = jnp.where(win, f, TRASH)
        plsc.store_scatter(tgt_v, [p // CHW, p % CHW], tgt)
        return 0

    lax.fori_loop(0, MROW // 16, compare, 0)

    def scatter(c, _):
        pltpu.sync_copy(pf_hbm.at[w, pl.ds(c * CHW, CHW), :], rows_v)
        pltpu.async_copy(rows_v, canvas_hbm.at[tgt_v.at[c]], sem).wait()
        return 0

    lax.fori_loop(0, NCH, scatter, 0)


_RB = 6912
_NJ = (NY * NX) // _RB


def _tc_body(canvas_ref, win_ref, out_ref):
    x = canvas_ref[...]
    wv = win_ref[0, 0, :]
    out_ref[0] = jnp.where((wv >= 0)[None, :], x.T, jnp.float32(0))


def _tc_finish(canvas, winner3):
    return pl.pallas_call(
        _tc_body,
        grid=(B, _NJ),
        in_specs=[
            pl.BlockSpec((_RB, C), lambda b, j: (b * _NJ + j, 0)),
            pl.BlockSpec((1, 1, _RB), lambda b, j: (b * _NJ + j, 0, 0)),
        ],
        out_specs=pl.BlockSpec((1, C, _RB), lambda b, j: (b, 0, j)),
        out_shape=jax.ShapeDtypeStruct((B, C, NY * NX), jnp.float32),
        compiler_params=pltpu.CompilerParams(
            dimension_semantics=("parallel", "parallel")),
    )(canvas, winner3)


def kernel(pillar_features, voxel_coords):
    coords3 = voxel_coords.reshape(NW, MP, 4)
    pf_pad = jnp.zeros((NW, MROW, C), jnp.float32)
    pf_pad = lax.dynamic_update_slice(
        pf_pad, pillar_features.reshape(NW, MP, C), (0, 0, 0))
    flat = _sc_flatten(coords3)
    winner = _sc_winner(flat)
    canvas = _sc_scatter(pf_pad, flat, winner)
    winner3 = winner.reshape(B * _NJ, 1, _RB)
    out = _tc_finish(canvas, winner3)
    return out.reshape(B, C, NY, NX)


# speedup vs baseline: 8.9159x; 2.3302x over previous
import functools

import jax
import jax.numpy as jnp
from jax import lax
from jax.experimental import pallas as pl
from jax.experimental.pallas import tpu as pltpu
from jax.experimental.pallas import tpu_sc as plsc

B = 4
C = 64
NX = 432
NY = 496
M = 80000
NPOS = B * NY * NX
NC = 2
NS = 16
NW = NC * NS
MP = M // NW
MPAD = 2512
NVEC = MPAD // 16
K = NPOS // NW
TRASH = NPOS
NCH = 20
CHW = 128
MROW = NCH * CHW
HY = NY // 2
XB = 16
NK = NX // XB
C2BLK = XB * HY
C2ROWS = C2BLK * (B * NK + 1)
CANVAS_ROWS = 2 * C2ROWS

_mesh = plsc.VectorSubcoreMesh(
    core_axis_name="c", subcore_axis_name="s", num_cores=NC, num_subcores=NS)

_sc_params = pltpu.CompilerParams(
    use_tc_tiling_on_sc=False, needs_layout_passes=False)


def _wid():
    return lax.axis_index("s") * NC + lax.axis_index("c")


def _iota():
    return lax.iota(jnp.int32, 16)


@functools.partial(
    pl.kernel,
    out_type=jax.ShapeDtypeStruct((NW * MPAD,), jnp.int32),
    mesh=_mesh,
    compiler_params=_sc_params,
    scratch_types=[
        pltpu.VMEM((MP, 4), jnp.int32),
        pltpu.VMEM((MPAD,), jnp.int32),
    ],
)
def _sc_flatten(coords_hbm, flat_hbm, coords_v, flat_v):
    w = _wid()
    pltpu.sync_copy(coords_hbm.at[w], coords_v)
    zeros = jnp.zeros((16,), jnp.int32)
    col_y = jnp.full((16,), 2, jnp.int32)
    col_x = jnp.full((16,), 3, jnp.int32)

    def body(i, _):
        p = i * 16 + _iota()
        valid = p < MP
        pc = jnp.minimum(p, MP - 1)
        b = plsc.load_gather(coords_v, [pc, zeros])
        y = plsc.load_gather(coords_v, [pc, col_y])
        x = plsc.load_gather(coords_v, [pc, col_x])
        g = b * (NY * NX) + x * NY + y
        flat_v[pl.ds(i * 16, 16)] = jnp.where(valid, g, -1)
        return 0

    lax.fori_loop(0, NVEC, body, 0)
    pltpu.sync_copy(flat_v, flat_hbm.at[pl.ds(w * MPAD, MPAD)])


@functools.partial(
    pl.kernel,
    out_type=jax.ShapeDtypeStruct((NPOS,), jnp.int32),
    mesh=_mesh,
    compiler_params=_sc_params,
    scratch_types=[
        pltpu.VMEM((NW * MPAD,), jnp.int32),
        pltpu.VMEM((K,), jnp.int32),
    ],
)
def _sc_winner(flat_hbm, winner_hbm, flat_v, seg_v):
    w = _wid()
    base = w * K
    pltpu.sync_copy(flat_hbm, flat_v)
    neg1 = jnp.full((16,), -1, jnp.int32)

    def init(i, _):
        seg_v[pl.ds(i * 16, 16)] = neg1
        return 0

    lax.fori_loop(0, K // 16, init, 0)

    def scan(j, _):
        f = flat_v[pl.ds(j * 16, 16)]
        local = f - base
        inr = local.astype(jnp.uint32) < jnp.uint32(K)
        cl = jnp.where(inr, local, 0)
        m = j * 16 + _iota()
        cur = plsc.load_gather(seg_v, [cl], mask=inr)
        new = jnp.maximum(cur, m)
        plsc.store_scatter(seg_v, [cl], new, mask=inr)
        return 0

    lax.fori_loop(0, (NW * MPAD) // 16, scan, 0)
    pltpu.sync_copy(seg_v, winner_hbm.at[pl.ds(base, K)])


@functools.partial(
    pl.kernel,
    out_type=jax.ShapeDtypeStruct((CANVAS_ROWS, C), jnp.float32),
    mesh=_mesh,
    compiler_params=_sc_params,
    scratch_types=[
        pltpu.VMEM((MROW,), jnp.int32),
        pltpu.VMEM((NCH, CHW), jnp.int32),
        pltpu.VMEM((NCH, CHW), jnp.int32),
        pltpu.VMEM((NCH, CHW), jnp.int32),
        pltpu.VMEM((CHW, C), jnp.float32),
        pltpu.SemaphoreType.DMA,
    ],
)
def _sc_scatter(pf_hbm, flat_hbm, winner_hbm, canvas_hbm,
                fidx_v, gidx_v, wv_v, tgt_v, rows_v, sem):
    w = _wid()
    fbase = w * MPAD
    pltpu.sync_copy(flat_hbm.at[pl.ds(fbase, MPAD)], fidx_v.at[pl.ds(0, MPAD)])

    def sanitize(i, _):
        p = i * 16 + _iota()
        valid = p < MP
        f = fidx_v[pl.ds(i * 16, 16)]
        fs = jnp.where(valid, f, 0)
        plsc.store_scatter(gidx_v, [p // CHW, p % CHW], fs)
        return 0

    lax.fori_loop(0, MROW // 16, sanitize, 0)

    def gather(c, _):
        pltpu.async_copy(winner_hbm.at[gidx_v.at[c]], wv_v.at[c], sem).wait()
        return 0

    lax.fori_loop(0, NCH, gather, 0)

    def compare(i, _):
        p = i * 16 + _iota()
        valid = p < MP
        f = fidx_v[pl.ds(i * 16, 16)]
        wv = plsc.load_gather(wv_v, [p // CHW, p % CHW])
        win = (wv == (fbase + p)) & valid
        b2 = f // (NY * NX)
        rem = f - b2 * (NY * NX)
        x = rem // NY
        y = rem - x * NY
        h = y // HY
        yl = y - h * HY
        s = (b2 * NX + x) * NY + yl * 2 + h
        tgt = jnp.where(win, s, TRASH)
        plsc.store_scatter(tgt_v, [p // CHW, p % CHW], tgt)
        return 0

    lax.fori_loop(0, MROW // 16, compare, 0)

    def scatter(c, _):
        pltpu.sync_copy(pf_hbm.at[w, pl.ds(c * CHW, CHW), :], rows_v)
        pltpu.async_copy(rows_v, canvas_hbm.at[tgt_v.at[c]], sem).wait()
        return 0

    lax.fori_loop(0, NCH, scatter, 0)


def _tc_body(c2_ref, win_ref, out_ref):
    wv = win_ref[0, 0, :]
    for xi in range(XB):
        wc = wv[xi * NY:(xi + 1) * NY]
        x2 = c2_ref[xi * HY:(xi + 1) * HY, :]
        av = x2[:, 0:C]
        bv = x2[:, C:2 * C]
        out_ref[0, :, xi, 0:HY] = jnp.where(
            (wc[0:HY] >= 0)[None, :], av.T, jnp.float32(0))
        out_ref[0, :, xi, HY:NY] = jnp.where(
            (wc[HY:NY] >= 0)[None, :], bv.T, jnp.float32(0))


def _tc_finish(canvas2, winner3):
    return pl.pallas_call(
        _tc_body,
        grid=(B, NK),
        in_specs=[
            pl.BlockSpec((C2BLK, 2 * C), lambda b, k: (b * NK + k, 0)),
            pl.BlockSpec((1, 1, XB * NY), lambda b, k: (b * NK + k, 0, 0)),
        ],
        out_specs=pl.BlockSpec((1, C, XB, NY), lambda b, k: (b, 0, k, 0)),
        out_shape=jax.ShapeDtypeStruct((B, C, NX, NY), jnp.float32),
        compiler_params=pltpu.CompilerParams(
            dimension_semantics=("parallel", "parallel")),
    )(canvas2, winner3)


def kernel(pillar_features, voxel_coords):
    coords3 = voxel_coords.reshape(NW, MP, 4)
    pf_pad = jnp.zeros((NW, MROW, C), jnp.float32)
    pf_pad = lax.dynamic_update_slice(
        pf_pad, pillar_features.reshape(NW, MP, C), (0, 0, 0))
    flat = _sc_flatten(coords3)
    winner = _sc_winner(flat)
    canvas = _sc_scatter(pf_pad, flat, winner)
    canvas2 = canvas.reshape(C2ROWS, 2 * C)
    winner3 = winner.reshape(B * NK, 1, XB * NY)
    out = _tc_finish(canvas2, winner3)
    return out.transpose(0, 1, 3, 2)


# speedup vs baseline: 14.0329x; 1.5739x over previous
import functools

import jax
import jax.numpy as jnp
from jax import lax
from jax.experimental import pallas as pl
from jax.experimental.pallas import tpu as pltpu
from jax.experimental.pallas import tpu_sc as plsc

B = 4
C = 64
NX = 432
NY = 496
M = 80000
NPOS = B * NY * NX
NC = 2
NS = 16
NW = NC * NS
MP = M // NW
MPAD = 2512
NVEC = MPAD // 16
K = NPOS // NW
NCH = 20
CHW = 128
MROW = NCH * CHW
CPC = 256
YR = NY - CPC
XB = 16
NK = NX // XB
C2BLK = XB * CPC
C2ROWS = C2BLK * (B * NK + 1)
CANVAS_ROWS = 2 * C2ROWS
TRASH = 2 * C2BLK * B * NK

_mesh = plsc.VectorSubcoreMesh(
    core_axis_name="c", subcore_axis_name="s", num_cores=NC, num_subcores=NS)

_sc_params = pltpu.CompilerParams(
    use_tc_tiling_on_sc=False, needs_layout_passes=False)


def _wid():
    return lax.axis_index("s") * NC + lax.axis_index("c")


def _iota():
    return lax.iota(jnp.int32, 16)


@functools.partial(
    pl.kernel,
    out_type=jax.ShapeDtypeStruct((NW * MPAD,), jnp.int32),
    mesh=_mesh,
    compiler_params=_sc_params,
    scratch_types=[
        pltpu.VMEM((MP, 4), jnp.int32),
        pltpu.VMEM((MPAD,), jnp.int32),
    ],
)
def _sc_flatten(coords_hbm, flat_hbm, coords_v, flat_v):
    w = _wid()
    pltpu.sync_copy(coords_hbm.at[w], coords_v)
    zeros = jnp.zeros((16,), jnp.int32)
    col_y = jnp.full((16,), 2, jnp.int32)
    col_x = jnp.full((16,), 3, jnp.int32)

    def body(i, _):
        p = i * 16 + _iota()
        valid = p < MP
        pc = jnp.minimum(p, MP - 1)
        b = plsc.load_gather(coords_v, [pc, zeros])
        y = plsc.load_gather(coords_v, [pc, col_y])
        x = plsc.load_gather(coords_v, [pc, col_x])
        g = b * (NY * NX) + x * NY + y
        flat_v[pl.ds(i * 16, 16)] = jnp.where(valid, g, -1)
        return 0

    lax.fori_loop(0, NVEC, body, 0)
    pltpu.sync_copy(flat_v, flat_hbm.at[pl.ds(w * MPAD, MPAD)])


@functools.partial(
    pl.kernel,
    out_type=jax.ShapeDtypeStruct((NPOS,), jnp.int32),
    mesh=_mesh,
    compiler_params=_sc_params,
    scratch_types=[
        pltpu.VMEM((NW * MPAD,), jnp.int32),
        pltpu.VMEM((K,), jnp.int32),
    ],
)
def _sc_winner(flat_hbm, winner_hbm, flat_v, seg_v):
    w = _wid()
    base = w * K
    pltpu.sync_copy(flat_hbm, flat_v)
    neg1 = jnp.full((16,), -1, jnp.int32)

    def init(i, _):
        seg_v[pl.ds(i * 16, 16)] = neg1
        return 0

    lax.fori_loop(0, K // 16, init, 0)

    def scan(j, _):
        f = flat_v[pl.ds(j * 16, 16)]
        local = f - base
        inr = local.astype(jnp.uint32) < jnp.uint32(K)
        cl = jnp.where(inr, local, 0)
        m = j * 16 + _iota()
        cur = plsc.load_gather(seg_v, [cl], mask=inr)
        new = jnp.maximum(cur, m)
        plsc.store_scatter(seg_v, [cl], new, mask=inr)
        return 0

    lax.fori_loop(0, (NW * MPAD) // 16, scan, 0)
    pltpu.sync_copy(seg_v, winner_hbm.at[pl.ds(base, K)])


@functools.partial(
    pl.kernel,
    out_type=jax.ShapeDtypeStruct((CANVAS_ROWS, C), jnp.float32),
    mesh=_mesh,
    compiler_params=_sc_params,
    scratch_types=[
        pltpu.VMEM((MROW,), jnp.int32),
        pltpu.VMEM((NCH, CHW), jnp.int32),
        pltpu.VMEM((NCH, CHW), jnp.int32),
        pltpu.VMEM((NCH, CHW), jnp.int32),
        pltpu.VMEM((CHW, C), jnp.float32),
        pltpu.SemaphoreType.DMA,
    ],
)
def _sc_scatter(pf_hbm, flat_hbm, winner_hbm, canvas_hbm,
                fidx_v, gidx_v, wv_v, tgt_v, rows_v, sem):
    w = _wid()
    fbase = w * MPAD
    pltpu.sync_copy(flat_hbm.at[pl.ds(fbase, MPAD)], fidx_v.at[pl.ds(0, MPAD)])

    def sanitize(i, _):
        p = i * 16 + _iota()
        valid = p < MP
        f = fidx_v[pl.ds(i * 16, 16)]
        fs = jnp.where(valid, f, 0)
        plsc.store_scatter(gidx_v, [p // CHW, p % CHW], fs)
        return 0

    lax.fori_loop(0, MROW // 16, sanitize, 0)

    def gather(c, _):
        pltpu.async_copy(winner_hbm.at[gidx_v.at[c]], wv_v.at[c], sem).wait()
        return 0

    lax.fori_loop(0, NCH, gather, 0)

    def compare(i, _):
        p = i * 16 + _iota()
        valid = p < MP
        f = fidx_v[pl.ds(i * 16, 16)]
        wv = plsc.load_gather(wv_v, [p // CHW, p % CHW])
        win = (wv == (fbase + p)) & valid
        b2 = f // (NY * NX)
        rem = f - b2 * (NY * NX)
        x = rem // NY
        y = rem - x * NY
        h = y // CPC
        yl = y - h * CPC
        s = (b2 * NX + x) * (2 * CPC) + yl * 2 + h
        tgt = jnp.where(win, s, TRASH)
        plsc.store_scatter(tgt_v, [p // CHW, p % CHW], tgt)
        return 0

    lax.fori_loop(0, MROW // 16, compare, 0)

    def scatter(c, _):
        pltpu.sync_copy(pf_hbm.at[w, pl.ds(c * CHW, CHW), :], rows_v)
        pltpu.async_copy(rows_v, canvas_hbm.at[tgt_v.at[c]], sem).wait()
        return 0

    lax.fori_loop(0, NCH, scatter, 0)


def _tc_body(c2_ref, win_ref, out_ref):
    wv = win_ref[0, 0, :]
    for xi in range(XB):
        wc = wv[xi * NY:(xi + 1) * NY]
        x2 = c2_ref[xi * CPC:(xi + 1) * CPC, :]
        xt = x2.T
        out_ref[0, :, xi, 0:CPC] = jnp.where(
            (wc[0:CPC] >= 0)[None, :], xt[0:C, :], jnp.float32(0))
        out_ref[0, :, xi, CPC:NY] = jnp.where(
            (wc[CPC:NY] >= 0)[None, :], xt[C:2 * C, 0:YR], jnp.float32(0))


def _tc_finish(canvas2, winner3):
    return pl.pallas_call(
        _tc_body,
        grid=(B, NK),
        in_specs=[
            pl.BlockSpec((C2BLK, 2 * C), lambda b, k: (b * NK + k, 0)),
            pl.BlockSpec((1, 1, XB * NY), lambda b, k: (b * NK + k, 0, 0)),
        ],
        out_specs=pl.BlockSpec((1, C, XB, NY), lambda b, k: (b, 0, k, 0)),
        out_shape=jax.ShapeDtypeStruct((B, C, NX, NY), jnp.float32),
        compiler_params=pltpu.CompilerParams(
            dimension_semantics=("parallel", "parallel")),
    )(canvas2, winner3)


def kernel(pillar_features, voxel_coords):
    coords3 = voxel_coords.reshape(NW, MP, 4)
    pf_pad = jnp.zeros((NW, MROW, C), jnp.float32)
    pf_pad = lax.dynamic_update_slice(
        pf_pad, pillar_features.reshape(NW, MP, C), (0, 0, 0))
    flat = _sc_flatten(coords3)
    winner = _sc_winner(flat)
    canvas = _sc_scatter(pf_pad, flat, winner)
    canvas2 = canvas.reshape(C2ROWS, 2 * C)
    winner3 = winner.reshape(B * NK, 1, XB * NY)
    out = _tc_finish(canvas2, winner3)
    return out.transpose(0, 1, 3, 2)
